# Initial kernel scaffold; baseline (speedup 1.0000x reference)
import jax
import jax.numpy as jnp
from jax.experimental import pallas as pl


def kernel(x, pos, edge_attr, edge_index, params):
    raise NotImplementedError("write your pallas kernel here")


# speedup vs baseline: 1.5520x; 1.5520x over previous
import functools

import jax
import jax.numpy as jnp
from jax import lax
from jax.experimental import pallas as pl
from jax.experimental.pallas import tpu as pltpu
from jax.experimental.pallas import tpu_sc as plsc

NN = 10000
EE = 320000
H = 128
NB = 2000
CS = 128
EPAD = 327680
EB = 2048
NCORE = 2
NSUB = 16
NW = NCORE * NSUB
EPW = EPAD // NW
NC = EPW // CS
NPAD = 10240
NPS = NPAD // NSUB


def _ln(h, g, be):
    mu = jnp.mean(h, axis=-1, keepdims=True)
    d = h - mu
    var = jnp.mean(d * d, axis=-1, keepdims=True)
    return d * lax.rsqrt(var + 1e-5) * g[None, :] + be[None, :]


def _dot(a, b):
    return jnp.dot(a, b, preferred_element_type=jnp.float32)


def _enc_node_body(x_ref, pos_ref, e0_ref, w0b_ref, w_ref, b_ref, o_ref):
    B = b_ref[...]
    oh = (x_ref[...] == lax.broadcasted_iota(jnp.int32, (1, 9), 1)).astype(jnp.float32)
    h = _dot(oh, e0_ref[...]) + _dot(pos_ref[...], w0b_ref[...]) + B[0][None, :]
    h = jnp.maximum(h, 0)
    W = w_ref[...]
    for i in range(3):
        h = _dot(h, W[i]) + B[i + 1][None, :]
        if i < 2:
            h = jnp.maximum(h, 0)
    o_ref[...] = _ln(h, B[4], B[5])


def _enc_edge_body(ea_ref, w0_ref, w_ref, b_ref, o_ref):
    B = b_ref[...]
    h = _dot(ea_ref[...], w0_ref[...]) + B[0][None, :]
    h = jnp.maximum(h, 0)
    W = w_ref[...]
    for i in range(3):
        h = _dot(h, W[i]) + B[i + 1][None, :]
        if i < 2:
            h = jnp.maximum(h, 0)
    o_ref[...] = _ln(h, B[4], B[5])


def _pq_body(nf_ref, w_ref, p_ref, q_ref):
    nf = nf_ref[...]
    W = w_ref[...]
    p_ref[...] = _dot(nf, W[0])
    q_ref[...] = _dot(nf, W[1])


def _edge_mlp_body(gp_ref, gq_ref, ef_ref, w_ref, b_ref, m_ref, efo_ref):
    ef = ef_ref[...]
    W = w_ref[...]
    B = b_ref[...]
    h = gp_ref[...] + gq_ref[...] + _dot(ef, W[0]) + B[0][None, :]
    h = jnp.maximum(h, 0)
    for i in range(1, 4):
        h = _dot(h, W[i]) + B[i][None, :]
        if i < 3:
            h = jnp.maximum(h, 0)
    m = _ln(h, B[4], B[5])
    m_ref[...] = m
    efo_ref[...] = m + ef


def _node_mlp_body(nf_ref, parts_ref, w_ref, b_ref, o_ref):
    nf = nf_ref[...]
    ag = parts_ref[0] + parts_ref[1]
    W = w_ref[...]
    B = b_ref[...]
    h = _dot(nf, W[0]) + _dot(ag, W[1]) + B[0][None, :]
    h = jnp.maximum(h, 0)
    for i in range(1, 4):
        h = _dot(h, W[i + 1]) + B[i][None, :]
        if i < 3:
            h = jnp.maximum(h, 0)
    o_ref[...] = _ln(h, B[4], B[5]) + nf


def _dec_body(nf_ref, w_ref, b_ref, w3_ref, b3_ref, o_ref):
    h = nf_ref[...]
    W = w_ref[...]
    B = b_ref[...]
    for i in range(3):
        h = jnp.maximum(_dot(h, W[i]) + B[i][None, :], 0)
    o_ref[...] = _dot(h, w3_ref[...]) + b3_ref[...]


def _full(shape):
    nd = len(shape)
    return pl.BlockSpec(shape, lambda i, _nd=nd: (0,) * _nd)


def _rows(block, width):
    return pl.BlockSpec((block, width), lambda i: (i, 0))


def _enc_node(x2d, pos, e0, w0b, w, b):
    return pl.pallas_call(
        _enc_node_body,
        grid=(NN // NB,),
        in_specs=[_rows(NB, 1), _rows(NB, 6), _full((9, H)), _full((6, H)),
                  _full((3, H, H)), _full((6, H))],
        out_specs=_rows(NB, H),
        out_shape=jax.ShapeDtypeStruct((NN, H), jnp.float32),
    )(x2d, pos, e0, w0b, w, b)


def _enc_edge(ea, w0, w, b):
    return pl.pallas_call(
        _enc_edge_body,
        grid=(EPAD // EB,),
        in_specs=[_rows(EB, 3), _full((3, H)), _full((3, H, H)), _full((6, H))],
        out_specs=_rows(EB, H),
        out_shape=jax.ShapeDtypeStruct((EPAD, H), jnp.float32),
    )(ea, w0, w, b)


def _pq(nf, w):
    return pl.pallas_call(
        _pq_body,
        grid=(NN // NB,),
        in_specs=[_rows(NB, H), _full((2, H, H))],
        out_specs=[_rows(NB, H), _rows(NB, H)],
        out_shape=[jax.ShapeDtypeStruct((NN, H), jnp.float32),
                   jax.ShapeDtypeStruct((NN, H), jnp.float32)],
    )(nf, w)


def _edge_mlp(gp, gq, ef, w, b):
    return pl.pallas_call(
        _edge_mlp_body,
        grid=(EPAD // EB,),
        in_specs=[_rows(EB, H), _rows(EB, H), _rows(EB, H),
                  _full((4, H, H)), _full((6, H))],
        out_specs=[_rows(EB, H), _rows(EB, H)],
        out_shape=[jax.ShapeDtypeStruct((EPAD, H), jnp.float32),
                   jax.ShapeDtypeStruct((EPAD, H), jnp.float32)],
    )(gp, gq, ef, w, b)


def _node_mlp(nf, parts, w, b):
    return pl.pallas_call(
        _node_mlp_body,
        grid=(NN // NB,),
        in_specs=[_rows(NB, H),
                  pl.BlockSpec((2, NB, H), lambda i: (0, i, 0)),
                  _full((5, H, H)), _full((6, H))],
        out_specs=_rows(NB, H),
        out_shape=jax.ShapeDtypeStruct((NN, H), jnp.float32),
    )(nf, parts, w, b)


def _dec(nf, w, b, w3, b3):
    return pl.pallas_call(
        _dec_body,
        grid=(NN // NB,),
        in_specs=[_rows(NB, H), _full((3, H, H)), _full((3, H)),
                  _full((H, 2)), _full((1, 2))],
        out_specs=_rows(NB, 2),
        out_shape=jax.ShapeDtypeStruct((NN, 2), jnp.float32),
    )(nf, w, b, w3, b3)


def _mesh():
    return plsc.VectorSubcoreMesh(core_axis_name="c", subcore_axis_name="s")


@jax.jit
def _sc_gather(p, q, dsti, srci):
    @functools.partial(
        pl.kernel,
        mesh=_mesh(),
        out_type=(jax.ShapeDtypeStruct((EPAD, H), jnp.float32),
                  jax.ShapeDtypeStruct((EPAD, H), jnp.float32)),
        scratch_types=[pltpu.VMEM((NC, CS), jnp.int32),
                       pltpu.VMEM((NC, CS), jnp.int32),
                       pltpu.VMEM((CS, H), jnp.float32),
                       pltpu.SemaphoreType.DMA],
    )
    def k(p_hbm, q_hbm, di_hbm, si_hbm, gp_hbm, gq_hbm, div, siv, rows, sem):
        wid = lax.axis_index("s") * NCORE + lax.axis_index("c")
        base = wid * EPW
        pltpu.sync_copy(di_hbm.at[wid], div)
        pltpu.sync_copy(si_hbm.at[wid], siv)

        @pl.loop(0, NC)
        def _(ci):
            off = base + ci * CS
            pltpu.async_copy(p_hbm.at[div.at[ci]], rows, sem).wait()
            pltpu.sync_copy(rows, gp_hbm.at[pl.ds(off, CS)])
            pltpu.async_copy(q_hbm.at[siv.at[ci]], rows, sem).wait()
            pltpu.sync_copy(rows, gq_hbm.at[pl.ds(off, CS)])

    return k(p, q, dsti, srci)


@jax.jit
def _sc_scatter(m, dsti, zeros):
    @functools.partial(
        pl.kernel,
        mesh=_mesh(),
        out_type=jax.ShapeDtypeStruct((NCORE, NPAD, H), jnp.float32),
        scratch_types=[pltpu.VMEM((NC, CS), jnp.int32),
                       pltpu.VMEM((CS, H), jnp.float32),
                       pltpu.VMEM_SHARED((NPAD, H), jnp.float32)],
    )
    def k(m_hbm, di_hbm, z_hbm, out_hbm, div, rows, shared):
        c = lax.axis_index("c")
        s = lax.axis_index("s")
        wid = s * NCORE + c
        base = wid * EPW

        @pl.when(s == 0)
        def _():
            pltpu.sync_copy(z_hbm, shared)

        plsc.subcore_barrier()
        pltpu.sync_copy(di_hbm.at[wid], div)

        @pl.loop(0, NC)
        def _(ci):
            pltpu.sync_copy(m_hbm.at[pl.ds(base + ci * CS, CS)], rows)
            pltpu.sync_copy(rows, shared.at[div.at[ci]], add=True)

        plsc.subcore_barrier()
        pltpu.sync_copy(shared.at[pl.ds(s * NPS, NPS)],
                        out_hbm.at[c].at[pl.ds(s * NPS, NPS)])

    return k(m, dsti, zeros)


def kernel(x, pos, edge_attr, edge_index, params):
    f32 = jnp.float32
    x2d = x.reshape(NN, 1).astype(jnp.int32)
    pos = pos.astype(f32)
    npadE = EPAD - EE
    dst = edge_index[1].astype(jnp.int32)
    src_ = edge_index[0].astype(jnp.int32)
    dsti_g = jnp.pad(dst, (0, npadE)).reshape(NW, NC, CS)
    srci_g = jnp.pad(src_, (0, npadE)).reshape(NW, NC, CS)
    padtgt = NN + (jnp.arange(npadE, dtype=jnp.int32) % (NPAD - NN))
    dsti_s = jnp.concatenate([dst, padtgt]).reshape(NW, NC, CS)
    ea_p = jnp.pad(edge_attr.astype(f32), ((0, npadE), (0, 0)))
    zeros = jnp.zeros((NPAD, H), f32)

    ni = params["node_in"]
    e0 = _dot(params["emb"], ni["W"][0][:16])
    nf = _enc_node(x2d, pos, e0, ni["W"][0][16:22],
                   jnp.stack(ni["W"][1:4]),
                   jnp.stack(ni["b"][0:4] + [ni["g"], ni["be"]]))

    ei = params["edge_in"]
    ef = _enc_edge(ea_p, ei["W"][0],
                   jnp.stack(ei["W"][1:4]),
                   jnp.stack(ei["b"][0:4] + [ei["g"], ei["be"]]))

    for lp in params["layers"]:
        epar, npar = lp["edge"], lp["node"]
        W1 = epar["W"][0]
        p, q = _pq(nf, jnp.stack([W1[:H], W1[H:2 * H]]))
        gp, gq = _sc_gather(p, q, dsti_g, srci_g)
        m, ef = _edge_mlp(
            gp, gq, ef,
            jnp.stack([W1[2 * H:]] + epar["W"][1:4]),
            jnp.stack(epar["b"][0:4] + [epar["g"], epar["be"]]))
        parts = _sc_scatter(m, dsti_s, zeros)
        Wn1 = npar["W"][0]
        nf = _node_mlp(
            nf, parts,
            jnp.stack([Wn1[:H], Wn1[H:]] + npar["W"][1:4]),
            jnp.stack(npar["b"][0:4] + [npar["g"], npar["be"]]))

    dp = params["dec"]
    return _dec(nf, jnp.stack(dp["W"][0:3]), jnp.stack(dp["b"][0:3]),
                dp["W"][3], dp["b"][3].reshape(1, 2))


# speedup vs baseline: 1.6981x; 1.0942x over previous
import functools

import jax
import jax.numpy as jnp
from jax import lax
from jax.experimental import pallas as pl
from jax.experimental.pallas import tpu as pltpu
from jax.experimental.pallas import tpu_sc as plsc

NN = 10000
EE = 320000
H = 128
NB = 2000
CS = 128
EPAD = 327680
EB = 2048
NCORE = 2
NSUB = 16
NW = NCORE * NSUB
EPW = EPAD // NW
NC = EPW // CS
NPAD = 10240
NPS = NPAD // NSUB


def _ln(h, g, be):
    mu = jnp.mean(h, axis=-1, keepdims=True)
    d = h - mu
    var = jnp.mean(d * d, axis=-1, keepdims=True)
    return d * lax.rsqrt(var + 1e-5) * g[None, :] + be[None, :]


def _dot(a, b):
    return jnp.dot(a, b, preferred_element_type=jnp.float32)


def _enc_node_body(x_ref, pos_ref, e0_ref, w0b_ref, w_ref, b_ref, o_ref):
    B = b_ref[...]
    oh = (x_ref[...] == lax.broadcasted_iota(jnp.int32, (1, 9), 1)).astype(jnp.float32)
    h = _dot(oh, e0_ref[...]) + _dot(pos_ref[...], w0b_ref[...]) + B[0][None, :]
    h = jnp.maximum(h, 0)
    W = w_ref[...]
    for i in range(3):
        h = _dot(h, W[i]) + B[i + 1][None, :]
        if i < 2:
            h = jnp.maximum(h, 0)
    o_ref[...] = _ln(h, B[4], B[5])


def _enc_edge_body(ea_ref, w0_ref, w_ref, b_ref, o_ref):
    B = b_ref[...]
    h = _dot(ea_ref[...], w0_ref[...]) + B[0][None, :]
    h = jnp.maximum(h, 0)
    W = w_ref[...]
    for i in range(3):
        h = _dot(h, W[i]) + B[i + 1][None, :]
        if i < 2:
            h = jnp.maximum(h, 0)
    o_ref[...] = _ln(h, B[4], B[5])


def _pq_body(nf_ref, w_ref, t_ref):
    t_ref[...] = _dot(nf_ref[...], w_ref[0])


def _edge_mlp_body(gp_ref, gq_ref, ef_ref, w_ref, b_ref, m_ref, efo_ref):
    ef = ef_ref[...]
    W = w_ref[...]
    B = b_ref[...]
    h = gp_ref[...] + gq_ref[...] + _dot(ef, W[0]) + B[0][None, :]
    h = jnp.maximum(h, 0)
    for i in range(1, 4):
        h = _dot(h, W[i]) + B[i][None, :]
        if i < 3:
            h = jnp.maximum(h, 0)
    m = _ln(h, B[4], B[5])
    m_ref[...] = m
    efo_ref[...] = m + ef


def _node_mlp_body(nf_ref, parts_ref, w_ref, b_ref, o_ref):
    nf = nf_ref[...]
    ag = parts_ref[0] + parts_ref[1]
    W = w_ref[...]
    B = b_ref[...]
    h = _dot(nf, W[0]) + _dot(ag, W[1]) + B[0][None, :]
    h = jnp.maximum(h, 0)
    for i in range(1, 4):
        h = _dot(h, W[i + 1]) + B[i][None, :]
        if i < 3:
            h = jnp.maximum(h, 0)
    o_ref[...] = _ln(h, B[4], B[5]) + nf


def _dec_body(nf_ref, w_ref, b_ref, w3_ref, b3_ref, o_ref):
    h = nf_ref[...]
    W = w_ref[...]
    B = b_ref[...]
    for i in range(3):
        h = jnp.maximum(_dot(h, W[i]) + B[i][None, :], 0)
    o_ref[...] = _dot(h, w3_ref[...]) + b3_ref[...]


def _full(shape):
    nd = len(shape)
    return pl.BlockSpec(shape, lambda i, _nd=nd: (0,) * _nd)


def _rows(block, width):
    return pl.BlockSpec((block, width), lambda i: (i, 0))


def _enc_node(x2d, pos, e0, w0b, w, b):
    return pl.pallas_call(
        _enc_node_body,
        grid=(NN // NB,),
        in_specs=[_rows(NB, 1), _rows(NB, 6), _full((9, H)), _full((6, H)),
                  _full((3, H, H)), _full((6, H))],
        out_specs=_rows(NB, H),
        out_shape=jax.ShapeDtypeStruct((NN, H), jnp.float32),
    )(x2d, pos, e0, w0b, w, b)


def _enc_edge(ea, w0, w, b):
    return pl.pallas_call(
        _enc_edge_body,
        grid=(EPAD // EB,),
        in_specs=[_rows(EB, 3), _full((3, H)), _full((3, H, H)), _full((6, H))],
        out_specs=_rows(EB, H),
        out_shape=jax.ShapeDtypeStruct((EPAD, H), jnp.float32),
    )(ea, w0, w, b)


def _pq(nf, w):
    return pl.pallas_call(
        _pq_body,
        grid=(2, NN // NB),
        in_specs=[pl.BlockSpec((NB, H), lambda j, i: (i, 0)),
                  pl.BlockSpec((1, H, H), lambda j, i: (j, 0, 0))],
        out_specs=pl.BlockSpec((NB, H), lambda j, i: (j * (NN // NB) + i, 0)),
        out_shape=jax.ShapeDtypeStruct((2 * NN, H), jnp.float32),
    )(nf, w)


def _edge_mlp(gpq, ef, w, b):
    nbt = EPW // EB
    gp_spec = pl.BlockSpec((EB, H), lambda i: ((i // nbt) * 2 * nbt + (i % nbt), 0))
    gq_spec = pl.BlockSpec((EB, H), lambda i: ((i // nbt) * 2 * nbt + nbt + (i % nbt), 0))
    return pl.pallas_call(
        _edge_mlp_body,
        grid=(EPAD // EB,),
        in_specs=[gp_spec, gq_spec, _rows(EB, H),
                  _full((4, H, H)), _full((6, H))],
        out_specs=[_rows(EB, H), _rows(EB, H)],
        out_shape=[jax.ShapeDtypeStruct((EPAD, H), jnp.float32),
                   jax.ShapeDtypeStruct((EPAD, H), jnp.float32)],
    )(gpq, gpq, ef, w, b)


def _node_mlp(nf, parts, w, b):
    return pl.pallas_call(
        _node_mlp_body,
        grid=(NN // NB,),
        in_specs=[_rows(NB, H),
                  pl.BlockSpec((2, NB, H), lambda i: (0, i, 0)),
                  _full((5, H, H)), _full((6, H))],
        out_specs=_rows(NB, H),
        out_shape=jax.ShapeDtypeStruct((NN, H), jnp.float32),
    )(nf, parts, w, b)


def _dec(nf, w, b, w3, b3):
    return pl.pallas_call(
        _dec_body,
        grid=(NN // NB,),
        in_specs=[_rows(NB, H), _full((3, H, H)), _full((3, H)),
                  _full((H, 2)), _full((1, 2))],
        out_specs=_rows(NB, 2),
        out_shape=jax.ShapeDtypeStruct((NN, 2), jnp.float32),
    )(nf, w, b, w3, b3)


def _mesh():
    return plsc.VectorSubcoreMesh(core_axis_name="c", subcore_axis_name="s")


KB = 5
NCT = 2 * NC


@jax.jit
def _sc_gather(t, idxc):
    @functools.partial(
        pl.kernel,
        mesh=_mesh(),
        out_type=jax.ShapeDtypeStruct((2 * EPAD, H), jnp.float32),
        scratch_types=[pltpu.VMEM((NCT, CS), jnp.int32)]
                      + [pltpu.VMEM((CS, H), jnp.float32) for _ in range(KB)]
                      + [pltpu.SemaphoreType.DMA for _ in range(2 * KB)],
    )
    def k(t_hbm, ix_hbm, out_hbm, ixv, *rest):
        bufs = rest[:KB]
        gsem = rest[KB:2 * KB]
        wsem = rest[2 * KB:]
        wid = lax.axis_index("s") * NCORE + lax.axis_index("c")
        obase = wid * NCT * CS
        pltpu.sync_copy(ix_hbm.at[wid], ixv)

        @pl.loop(0, NCT, step=KB)
        def _(ci):
            hs = []
            for b in range(KB):
                hs.append(pltpu.async_copy(t_hbm.at[ixv.at[ci + b]], bufs[b],
                                           gsem[b]))
            ws = []
            for b in range(KB):
                hs[b].wait()
                ws.append(pltpu.async_copy(
                    bufs[b], out_hbm.at[pl.ds(obase + (ci + b) * CS, CS)],
                    wsem[b]))
            for w in ws:
                w.wait()

    return k(t, idxc)


KS = 4


@jax.jit
def _sc_scatter(m, dsti, zeros):
    @functools.partial(
        pl.kernel,
        mesh=_mesh(),
        out_type=jax.ShapeDtypeStruct((NCORE, NPAD, H), jnp.float32),
        scratch_types=[pltpu.VMEM((NC, CS), jnp.int32),
                       pltpu.VMEM((CS, H), jnp.float32),
                       pltpu.VMEM_SHARED((NPAD, H), jnp.float32)],
    )
    def k(m_hbm, di_hbm, z_hbm, out_hbm, div, rows, shared):
        c = lax.axis_index("c")
        s = lax.axis_index("s")
        wid = s * NCORE + c
        base = wid * EPW

        @pl.when(s == 0)
        def _():
            pltpu.sync_copy(z_hbm, shared)

        plsc.subcore_barrier()
        pltpu.sync_copy(di_hbm.at[wid], div)

        @pl.loop(0, NC)
        def _(ci):
            pltpu.sync_copy(m_hbm.at[pl.ds(base + ci * CS, CS)], rows)
            pltpu.sync_copy(rows, shared.at[div.at[ci]], add=True)

        plsc.subcore_barrier()
        pltpu.sync_copy(shared.at[pl.ds(s * NPS, NPS)],
                        out_hbm.at[c].at[pl.ds(s * NPS, NPS)])

    return k(m, dsti, zeros)


def kernel(x, pos, edge_attr, edge_index, params):
    f32 = jnp.float32
    x2d = x.reshape(NN, 1).astype(jnp.int32)
    pos = pos.astype(f32)
    npadE = EPAD - EE
    dst = edge_index[1].astype(jnp.int32)
    src_ = edge_index[0].astype(jnp.int32)
    dsti_g = jnp.pad(dst, (0, npadE)).reshape(NW, NC, CS)
    srci_g = jnp.pad(src_, (0, npadE)).reshape(NW, NC, CS)
    idxc = jnp.concatenate([dsti_g, srci_g + NN], axis=1)
    padtgt = NN + (jnp.arange(npadE, dtype=jnp.int32) % (NPAD - NN))
    dsti_s = jnp.concatenate([dst, padtgt]).reshape(NW, NC, CS)
    ea_p = jnp.pad(edge_attr.astype(f32), ((0, npadE), (0, 0)))
    zeros = jnp.zeros((NPAD, H), f32)

    ni = params["node_in"]
    e0 = _dot(params["emb"], ni["W"][0][:16])
    nf = _enc_node(x2d, pos, e0, ni["W"][0][16:22],
                   jnp.stack(ni["W"][1:4]),
                   jnp.stack(ni["b"][0:4] + [ni["g"], ni["be"]]))

    ei = params["edge_in"]
    ef = _enc_edge(ea_p, ei["W"][0],
                   jnp.stack(ei["W"][1:4]),
                   jnp.stack(ei["b"][0:4] + [ei["g"], ei["be"]]))

    for lp in params["layers"]:
        epar, npar = lp["edge"], lp["node"]
        W1 = epar["W"][0]
        t = _pq(nf, jnp.stack([W1[:H], W1[H:2 * H]]))
        gpq = _sc_gather(t, idxc)
        m, ef = _edge_mlp(
            gpq, ef,
            jnp.stack([W1[2 * H:]] + epar["W"][1:4]),
            jnp.stack(epar["b"][0:4] + [epar["g"], epar["be"]]))
        parts = _sc_scatter(m, dsti_s, zeros)
        Wn1 = npar["W"][0]
        nf = _node_mlp(
            nf, parts,
            jnp.stack([Wn1[:H], Wn1[H:]] + npar["W"][1:4]),
            jnp.stack(npar["b"][0:4] + [npar["g"], npar["be"]]))

    dp = params["dec"]
    return _dec(nf, jnp.stack(dp["W"][0:3]), jnp.stack(dp["b"][0:3]),
                dp["W"][3], dp["b"][3].reshape(1, 2))


# speedup vs baseline: 3.7764x; 2.2239x over previous
import functools

import jax
import jax.numpy as jnp
from jax import lax
from jax.experimental import pallas as pl
from jax.experimental.pallas import tpu as pltpu
from jax.experimental.pallas import tpu_sc as plsc

NN = 10000
EE = 320000
H = 128
NB = 2000
CS = 128
EPAD = 327680
EB = 2048
NCORE = 2
NSUB = 16
NW = NCORE * NSUB
EPW = EPAD // NW
NC = EPW // CS
NPAD = 10240
NPS = NPAD // NSUB


def _ln(h, g, be):
    mu = jnp.mean(h, axis=-1, keepdims=True)
    d = h - mu
    var = jnp.mean(d * d, axis=-1, keepdims=True)
    return d * lax.rsqrt(var + 1e-5) * g[None, :] + be[None, :]


def _dot(a, b):
    return jnp.dot(a, b, preferred_element_type=jnp.float32)


def _enc_node_body(x_ref, pos_ref, e0_ref, w0b_ref, w_ref, b_ref, o_ref):
    B = b_ref[...]
    oh = (x_ref[...] == lax.broadcasted_iota(jnp.int32, (1, 9), 1)).astype(jnp.float32)
    h = _dot(oh, e0_ref[...]) + _dot(pos_ref[...], w0b_ref[...]) + B[0][None, :]
    h = jnp.maximum(h, 0)
    W = w_ref[...]
    for i in range(3):
        h = _dot(h, W[i]) + B[i + 1][None, :]
        if i < 2:
            h = jnp.maximum(h, 0)
    o_ref[...] = _ln(h, B[4], B[5])


def _enc_edge_body(ea_ref, w0_ref, w_ref, b_ref, o_ref):
    B = b_ref[...]
    h = _dot(ea_ref[...], w0_ref[...]) + B[0][None, :]
    h = jnp.maximum(h, 0)
    W = w_ref[...]
    for i in range(3):
        h = _dot(h, W[i]) + B[i + 1][None, :]
        if i < 2:
            h = jnp.maximum(h, 0)
    o_ref[...] = _ln(h, B[4], B[5])


def _pq_body(nf_ref, w_ref, t_ref):
    t_ref[...] = _dot(nf_ref[...], w_ref[0])


def _edge_mlp_body(gp_ref, gq_ref, ef_ref, w_ref, b_ref, m_ref, efo_ref):
    ef = ef_ref[...]
    W = w_ref[...]
    B = b_ref[...]
    h = gp_ref[0] + gq_ref[0] + _dot(ef, W[0]) + B[0][None, :]
    h = jnp.maximum(h, 0)
    for i in range(1, 4):
        h = _dot(h, W[i]) + B[i][None, :]
        if i < 3:
            h = jnp.maximum(h, 0)
    m = _ln(h, B[4], B[5])
    m_ref[...] = m
    efo_ref[...] = m + ef


def _node_mlp_body(nf_ref, parts_ref, w_ref, b_ref, o_ref):
    nf = nf_ref[...]
    ag = parts_ref[0] + parts_ref[1]
    W = w_ref[...]
    B = b_ref[...]
    h = _dot(nf, W[0]) + _dot(ag, W[1]) + B[0][None, :]
    h = jnp.maximum(h, 0)
    for i in range(1, 4):
        h = _dot(h, W[i + 1]) + B[i][None, :]
        if i < 3:
            h = jnp.maximum(h, 0)
    o_ref[...] = _ln(h, B[4], B[5]) + nf


def _dec_body(nf_ref, w_ref, b_ref, w3_ref, b3_ref, o_ref):
    h = nf_ref[...]
    W = w_ref[...]
    B = b_ref[...]
    for i in range(3):
        h = jnp.maximum(_dot(h, W[i]) + B[i][None, :], 0)
    o_ref[...] = _dot(h, w3_ref[...]) + b3_ref[...]


def _full(shape):
    nd = len(shape)
    return pl.BlockSpec(shape, lambda i, _nd=nd: (0,) * _nd)


def _rows(block, width):
    return pl.BlockSpec((block, width), lambda i: (i, 0))


def _enc_node(x2d, pos, e0, w0b, w, b):
    return pl.pallas_call(
        _enc_node_body,
        grid=(NN // NB,),
        in_specs=[_rows(NB, 1), _rows(NB, 6), _full((9, H)), _full((6, H)),
                  _full((3, H, H)), _full((6, H))],
        out_specs=_rows(NB, H),
        out_shape=jax.ShapeDtypeStruct((NN, H), jnp.float32),
    )(x2d, pos, e0, w0b, w, b)


def _enc_edge(ea, w0, w, b):
    return pl.pallas_call(
        _enc_edge_body,
        grid=(EPAD // EB,),
        in_specs=[_rows(EB, 3), _full((3, H)), _full((3, H, H)), _full((6, H))],
        out_specs=_rows(EB, H),
        out_shape=jax.ShapeDtypeStruct((EPAD, H), jnp.float32),
    )(ea, w0, w, b)


def _pq(nf, w):
    return pl.pallas_call(
        _pq_body,
        grid=(2, NN // NB),
        in_specs=[pl.BlockSpec((NB, H), lambda j, i: (i, 0)),
                  pl.BlockSpec((1, H, H), lambda j, i: (j, 0, 0))],
        out_specs=pl.BlockSpec((NB, H), lambda j, i: (j * (NN // NB) + i, 0)),
        out_shape=jax.ShapeDtypeStruct((2 * NN, H), jnp.float32),
    )(nf, w)


def _edge_mlp(gpq, ef, w, b):
    gp_spec = pl.BlockSpec((1, EB, H), lambda i: (0, i, 0))
    gq_spec = pl.BlockSpec((1, EB, H), lambda i: (1, i, 0))
    return pl.pallas_call(
        _edge_mlp_body,
        grid=(EPAD // EB,),
        in_specs=[gp_spec, gq_spec, _rows(EB, H),
                  _full((4, H, H)), _full((6, H))],
        out_specs=[_rows(EB, H), _rows(EB, H)],
        out_shape=[jax.ShapeDtypeStruct((EPAD, H), jnp.float32),
                   jax.ShapeDtypeStruct((EPAD, H), jnp.float32)],
    )(gpq, gpq, ef, w, b)


def _node_mlp(nf, parts, w, b):
    return pl.pallas_call(
        _node_mlp_body,
        grid=(NN // NB,),
        in_specs=[_rows(NB, H),
                  pl.BlockSpec((2, NB, H), lambda i: (0, i, 0)),
                  _full((5, H, H)), _full((6, H))],
        out_specs=_rows(NB, H),
        out_shape=jax.ShapeDtypeStruct((NN, H), jnp.float32),
    )(nf, parts, w, b)


def _dec(nf, w, b, w3, b3):
    return pl.pallas_call(
        _dec_body,
        grid=(NN // NB,),
        in_specs=[_rows(NB, H), _full((3, H, H)), _full((3, H)),
                  _full((H, 2)), _full((1, 2))],
        out_specs=_rows(NB, 2),
        out_shape=jax.ShapeDtypeStruct((NN, 2), jnp.float32),
    )(nf, w, b, w3, b3)


def _mesh():
    return plsc.VectorSubcoreMesh(core_axis_name="c", subcore_axis_name="s")


GB = 8
NCW = EPAD // NSUB // CS


@jax.jit
def _sc_gather(t, idxg):
    @functools.partial(
        pl.kernel,
        mesh=_mesh(),
        out_type=jax.ShapeDtypeStruct((2, EPAD, H), jnp.float32),
        scratch_types=[pltpu.VMEM((GB, CS), jnp.int32),
                       pltpu.VMEM((CS, H), jnp.float32),
                       pltpu.VMEM((CS, H), jnp.float32),
                       pltpu.VMEM_SHARED((NN, H), jnp.float32),
                       pltpu.SemaphoreType.DMA,
                       pltpu.SemaphoreType.DMA],
    )
    def k(t_hbm, ix_hbm, out_hbm, ixv, buf0, buf1, table, w0, w1):
        c = lax.axis_index("c")
        s = lax.axis_index("s")

        @pl.when(s == 0)
        def _():
            pltpu.sync_copy(t_hbm.at[pl.ds(c * NN, NN)], table)

        plsc.subcore_barrier()
        obase = s * NCW * CS
        bufs = (buf0, buf1)
        wsem = (w0, w1)

        @pl.loop(0, NCW, step=GB)
        def _(cb):
            pltpu.sync_copy(ix_hbm.at[c].at[s].at[pl.ds(cb, GB)], ixv)
            hw = [None, None]
            for j in range(GB):
                b = j % 2
                if hw[b] is not None:
                    hw[b].wait()
                pltpu.sync_copy(table.at[ixv.at[j]], bufs[b])
                hw[b] = pltpu.async_copy(
                    bufs[b],
                    out_hbm.at[c].at[pl.ds(obase + (cb + j) * CS, CS)],
                    wsem[b])
            hw[0].wait()
            hw[1].wait()

    return k(t, idxg)


KS = 4


@jax.jit
def _sc_scatter(m, dsti, zeros):
    @functools.partial(
        pl.kernel,
        mesh=_mesh(),
        out_type=jax.ShapeDtypeStruct((NCORE, NPAD, H), jnp.float32),
        scratch_types=[pltpu.VMEM((NC, CS), jnp.int32),
                       pltpu.VMEM((CS, H), jnp.float32),
                       pltpu.VMEM_SHARED((NPAD, H), jnp.float32)],
    )
    def k(m_hbm, di_hbm, z_hbm, out_hbm, div, rows, shared):
        c = lax.axis_index("c")
        s = lax.axis_index("s")
        wid = s * NCORE + c
        base = wid * EPW

        @pl.when(s == 0)
        def _():
            pltpu.sync_copy(z_hbm, shared)

        plsc.subcore_barrier()
        pltpu.sync_copy(di_hbm.at[wid], div)

        @pl.loop(0, NC)
        def _(ci):
            pltpu.sync_copy(m_hbm.at[pl.ds(base + ci * CS, CS)], rows)
            pltpu.sync_copy(rows, shared.at[div.at[ci]], add=True)

        plsc.subcore_barrier()
        pltpu.sync_copy(shared.at[pl.ds(s * NPS, NPS)],
                        out_hbm.at[c].at[pl.ds(s * NPS, NPS)])

    return k(m, dsti, zeros)


def kernel(x, pos, edge_attr, edge_index, params):
    f32 = jnp.float32
    x2d = x.reshape(NN, 1).astype(jnp.int32)
    pos = pos.astype(f32)
    npadE = EPAD - EE
    dst = edge_index[1].astype(jnp.int32)
    src_ = edge_index[0].astype(jnp.int32)
    dsti_g = jnp.pad(dst, (0, npadE)).reshape(NSUB, NCW, CS)
    srci_g = jnp.pad(src_, (0, npadE)).reshape(NSUB, NCW, CS)
    idxg = jnp.stack([dsti_g, srci_g])
    padtgt = NN + (jnp.arange(npadE, dtype=jnp.int32) % (NPAD - NN))
    dsti_s = jnp.concatenate([dst, padtgt]).reshape(NW, NC, CS)
    ea_p = jnp.pad(edge_attr.astype(f32), ((0, npadE), (0, 0)))
    zeros = jnp.zeros((NPAD, H), f32)

    ni = params["node_in"]
    e0 = _dot(params["emb"], ni["W"][0][:16])
    nf = _enc_node(x2d, pos, e0, ni["W"][0][16:22],
                   jnp.stack(ni["W"][1:4]),
                   jnp.stack(ni["b"][0:4] + [ni["g"], ni["be"]]))

    ei = params["edge_in"]
    ef = _enc_edge(ea_p, ei["W"][0],
                   jnp.stack(ei["W"][1:4]),
                   jnp.stack(ei["b"][0:4] + [ei["g"], ei["be"]]))

    for lp in params["layers"]:
        epar, npar = lp["edge"], lp["node"]
        W1 = epar["W"][0]
        t = _pq(nf, jnp.stack([W1[:H], W1[H:2 * H]]))
        gpq = _sc_gather(t, idxg)
        m, ef = _edge_mlp(
            gpq, ef,
            jnp.stack([W1[2 * H:]] + epar["W"][1:4]),
            jnp.stack(epar["b"][0:4] + [epar["g"], epar["be"]]))
        parts = _sc_scatter(m, dsti_s, zeros)
        Wn1 = npar["W"][0]
        nf = _node_mlp(
            nf, parts,
            jnp.stack([Wn1[:H], Wn1[H:]] + npar["W"][1:4]),
            jnp.stack(npar["b"][0:4] + [npar["g"], npar["be"]]))

    dp = params["dec"]
    return _dec(nf, jnp.stack(dp["W"][0:3]), jnp.stack(dp["b"][0:3]),
                dp["W"][3], dp["b"][3].reshape(1, 2))


# speedup vs baseline: 4.4348x; 1.1744x over previous
import functools

import jax
import jax.numpy as jnp
from jax import lax
from jax.experimental import pallas as pl
from jax.experimental.pallas import tpu as pltpu
from jax.experimental.pallas import tpu_sc as plsc

NN = 10000
EE = 320000
H = 128
NB = 2000
CS = 128
EPAD = 327680
NE2 = EPAD // 2
EB = 2048
NCORE = 2
NSUB = 16
NW = NCORE * NSUB
NCW = NE2 // NSUB // CS
NCS = NE2 // NW // CS
NPAD = 10240
NPS = NPAD // NSUB
GB = 8


def _ln(h, g, be):
    mu = jnp.mean(h, axis=-1, keepdims=True)
    d = h - mu
    var = jnp.mean(d * d, axis=-1, keepdims=True)
    return d * lax.rsqrt(var + 1e-5) * g[None, :] + be[None, :]


def _dot(a, b):
    return jnp.dot(a, b, preferred_element_type=jnp.float32)


def _enc_node_body(x_ref, pos_ref, e0_ref, w0b_ref, w_ref, b_ref, o_ref):
    B = b_ref[...]
    oh = (x_ref[...] == lax.broadcasted_iota(jnp.int32, (1, 9), 1)).astype(jnp.float32)
    h = _dot(oh, e0_ref[...]) + _dot(pos_ref[...], w0b_ref[...]) + B[0][None, :]
    h = jnp.maximum(h, 0)
    W = w_ref[...]
    for i in range(3):
        h = _dot(h, W[i]) + B[i + 1][None, :]
        if i < 2:
            h = jnp.maximum(h, 0)
    o_ref[...] = _ln(h, B[4], B[5])


def _enc_edge_body(ea_ref, w0_ref, w_ref, b_ref, o_ref):
    B = b_ref[...]
    h = _dot(ea_ref[...], w0_ref[...]) + B[0][None, :]
    h = jnp.maximum(h, 0)
    W = w_ref[...]
    for i in range(3):
        h = _dot(h, W[i]) + B[i + 1][None, :]
        if i < 2:
            h = jnp.maximum(h, 0)
    o_ref[...] = _ln(h, B[4], B[5])


def _pq_body(nf_ref, w_ref, t_ref):
    t_ref[...] = _dot(nf_ref[...], w_ref[0])


def _edge_mlp_body(gp_ref, gq_ref, ef_ref, w_ref, b_ref, m_ref, efo_ref):
    ef = ef_ref[...]
    W = w_ref[...]
    B = b_ref[...]
    h = gp_ref[0] + gq_ref[0] + _dot(ef, W[0]) + B[0][None, :]
    h = jnp.maximum(h, 0)
    for i in range(1, 4):
        h = _dot(h, W[i]) + B[i][None, :]
        if i < 3:
            h = jnp.maximum(h, 0)
    m = _ln(h, B[4], B[5])
    m_ref[...] = m
    efo_ref[...] = m + ef


def _node_mlp_body(nf_ref, pa_ref, pb_ref, w_ref, b_ref, o_ref):
    nf = nf_ref[...]
    ag = pa_ref[0] + pa_ref[1] + pb_ref[0] + pb_ref[1]
    W = w_ref[...]
    B = b_ref[...]
    h = _dot(nf, W[0]) + _dot(ag, W[1]) + B[0][None, :]
    h = jnp.maximum(h, 0)
    for i in range(1, 4):
        h = _dot(h, W[i + 1]) + B[i][None, :]
        if i < 3:
            h = jnp.maximum(h, 0)
    o_ref[...] = _ln(h, B[4], B[5]) + nf


def _dec_body(nf_ref, w_ref, b_ref, w3_ref, b3_ref, o_ref):
    h = nf_ref[...]
    W = w_ref[...]
    B = b_ref[...]
    for i in range(3):
        h = jnp.maximum(_dot(h, W[i]) + B[i][None, :], 0)
    o_ref[...] = _dot(h, w3_ref[...]) + b3_ref[...]


def _full(shape):
    nd = len(shape)
    return pl.BlockSpec(shape, lambda i, _nd=nd: (0,) * _nd)


def _rows(block, width):
    return pl.BlockSpec((block, width), lambda i: (i, 0))


def _enc_node(x2d, pos, e0, w0b, w, b):
    return pl.pallas_call(
        _enc_node_body,
        grid=(NN // NB,),
        in_specs=[_rows(NB, 1), _rows(NB, 6), _full((9, H)), _full((6, H)),
                  _full((3, H, H)), _full((6, H))],
        out_specs=_rows(NB, H),
        out_shape=jax.ShapeDtypeStruct((NN, H), jnp.float32),
    )(x2d, pos, e0, w0b, w, b)


def _enc_edge(ea, w0, w, b):
    return pl.pallas_call(
        _enc_edge_body,
        grid=(NE2 // EB,),
        in_specs=[_rows(EB, 3), _full((3, H)), _full((3, H, H)), _full((6, H))],
        out_specs=_rows(EB, H),
        out_shape=jax.ShapeDtypeStruct((NE2, H), jnp.float32),
    )(ea, w0, w, b)


def _pq(nf, w):
    return pl.pallas_call(
        _pq_body,
        grid=(2, NN // NB),
        in_specs=[pl.BlockSpec((NB, H), lambda j, i: (i, 0)),
                  pl.BlockSpec((1, H, H), lambda j, i: (j, 0, 0))],
        out_specs=pl.BlockSpec((NB, H), lambda j, i: (j * (NN // NB) + i, 0)),
        out_shape=jax.ShapeDtypeStruct((2 * NN, H), jnp.float32),
    )(nf, w)


def _edge_mlp(gpq, ef, w, b):
    gp_spec = pl.BlockSpec((1, EB, H), lambda i: (0, i, 0))
    gq_spec = pl.BlockSpec((1, EB, H), lambda i: (1, i, 0))
    return pl.pallas_call(
        _edge_mlp_body,
        grid=(NE2 // EB,),
        in_specs=[gp_spec, gq_spec, _rows(EB, H),
                  _full((4, H, H)), _full((6, H))],
        out_specs=[_rows(EB, H), _rows(EB, H)],
        out_shape=[jax.ShapeDtypeStruct((NE2, H), jnp.float32),
                   jax.ShapeDtypeStruct((NE2, H), jnp.float32)],
    )(gpq, gpq, ef, w, b)


def _node_mlp(nf, pa, pb, w, b):
    pspec = pl.BlockSpec((2, NB, H), lambda i: (0, i, 0))
    return pl.pallas_call(
        _node_mlp_body,
        grid=(NN // NB,),
        in_specs=[_rows(NB, H), pspec, pspec, _full((5, H, H)), _full((6, H))],
        out_specs=_rows(NB, H),
        out_shape=jax.ShapeDtypeStruct((NN, H), jnp.float32),
    )(nf, pa, pb, w, b)


def _dec(nf, w, b, w3, b3):
    return pl.pallas_call(
        _dec_body,
        grid=(NN // NB,),
        in_specs=[_rows(NB, H), _full((3, H, H)), _full((3, H)),
                  _full((H, 2)), _full((1, 2))],
        out_specs=_rows(NB, 2),
        out_shape=jax.ShapeDtypeStruct((NN, 2), jnp.float32),
    )(nf, w, b, w3, b3)


def _mesh():
    return plsc.VectorSubcoreMesh(core_axis_name="c", subcore_axis_name="s")


def _sc_gather(t, idxg):
    @functools.partial(
        pl.kernel,
        mesh=_mesh(),
        out_type=jax.ShapeDtypeStruct((2, NE2, H), jnp.float32),
        scratch_types=[pltpu.VMEM((GB, CS), jnp.int32),
                       pltpu.VMEM((CS, H), jnp.float32),
                       pltpu.VMEM((CS, H), jnp.float32),
                       pltpu.VMEM_SHARED((NN, H), jnp.float32),
                       pltpu.SemaphoreType.DMA,
                       pltpu.SemaphoreType.DMA],
    )
    def k(t_hbm, ix_hbm, out_hbm, ixv, buf0, buf1, table, w0, w1):
        c = lax.axis_index("c")
        s = lax.axis_index("s")

        @pl.when(s == 0)
        def _():
            pltpu.sync_copy(t_hbm.at[pl.ds(c * NN, NN)], table)

        plsc.subcore_barrier()
        obase = s * NCW * CS
        bufs = (buf0, buf1)
        wsem = (w0, w1)

        @pl.loop(0, NCW, step=GB)
        def _(cb):
            pltpu.sync_copy(ix_hbm.at[c].at[s].at[pl.ds(cb, GB)], ixv)
            hw = [None, None]
            for j in range(GB):
                b = j % 2
                if hw[b] is not None:
                    hw[b].wait()
                pltpu.sync_copy(table.at[ixv.at[j]], bufs[b])
                hw[b] = pltpu.async_copy(
                    bufs[b],
                    out_hbm.at[c].at[pl.ds(obase + (cb + j) * CS, CS)],
                    wsem[b])
            hw[0].wait()
            hw[1].wait()

    return k(t, idxg)


def _sc_scatter(m, dsti, zeros):
    @functools.partial(
        pl.kernel,
        mesh=_mesh(),
        out_type=jax.ShapeDtypeStruct((NCORE, NPAD, H), jnp.float32),
        scratch_types=[pltpu.VMEM((GB, CS), jnp.int32),
                       pltpu.VMEM((CS, H), jnp.float32),
                       pltpu.VMEM((CS, H), jnp.float32),
                       pltpu.VMEM_SHARED((NPAD, H), jnp.float32),
                       pltpu.SemaphoreType.DMA,
                       pltpu.SemaphoreType.DMA],
    )
    def k(m_hbm, di_hbm, z_hbm, out_hbm, ixv, buf0, buf1, shared, r0, r1):
        c = lax.axis_index("c")
        s = lax.axis_index("s")
        wid = s * NCORE + c
        base = wid * NCS * CS

        @pl.when(s == 0)
        def _():
            pltpu.sync_copy(z_hbm, shared)

        plsc.subcore_barrier()
        bufs = (buf0, buf1)
        rsem = (r0, r1)

        @pl.loop(0, NCS, step=GB)
        def _(cb):
            pltpu.sync_copy(di_hbm.at[wid].at[pl.ds(cb, GB)], ixv)
            hr = [None, None]
            hr[0] = pltpu.async_copy(
                m_hbm.at[pl.ds(base + cb * CS, CS)], bufs[0], rsem[0])
            for j in range(GB):
                b = j % 2
                if j + 1 < GB:
                    hr[1 - b] = pltpu.async_copy(
                        m_hbm.at[pl.ds(base + (cb + j + 1) * CS, CS)],
                        bufs[1 - b], rsem[1 - b])
                hr[b].wait()
                pltpu.sync_copy(bufs[b], shared.at[ixv.at[j]], add=True)

        plsc.subcore_barrier()
        pltpu.sync_copy(shared.at[pl.ds(s * NPS, NPS)],
                        out_hbm.at[c].at[pl.ds(s * NPS, NPS)])

    return k(m, dsti, zeros)


def kernel(x, pos, edge_attr, edge_index, params):
    f32 = jnp.float32
    x2d = x.reshape(NN, 1).astype(jnp.int32)
    pos = pos.astype(f32)
    npadE = EPAD - EE
    dst = edge_index[1].astype(jnp.int32)
    src_ = edge_index[0].astype(jnp.int32)
    dst_p = jnp.pad(dst, (0, npadE))
    src_p = jnp.pad(src_, (0, npadE))
    idxg = [jnp.stack([dst_p[h * NE2:(h + 1) * NE2].reshape(NSUB, NCW, CS),
                       src_p[h * NE2:(h + 1) * NE2].reshape(NSUB, NCW, CS)])
            for h in range(2)]
    padtgt = NN + (jnp.arange(npadE, dtype=jnp.int32) % (NPAD - NN))
    dst_s = jnp.concatenate([dst, padtgt])
    dsti_s = [dst_s[h * NE2:(h + 1) * NE2].reshape(NW, NCS, CS)
              for h in range(2)]
    ea_p = jnp.pad(edge_attr.astype(f32), ((0, npadE), (0, 0)))
    zeros = jnp.zeros((NPAD, H), f32)

    ni = params["node_in"]
    e0 = _dot(params["emb"], ni["W"][0][:16])
    nf = _enc_node(x2d, pos, e0, ni["W"][0][16:22],
                   jnp.stack(ni["W"][1:4]),
                   jnp.stack(ni["b"][0:4] + [ni["g"], ni["be"]]))

    ei = params["edge_in"]
    ew0, eww, ewb = (ei["W"][0], jnp.stack(ei["W"][1:4]),
                     jnp.stack(ei["b"][0:4] + [ei["g"], ei["be"]]))
    ef = [_enc_edge(ea_p[h * NE2:(h + 1) * NE2], ew0, eww, ewb)
          for h in range(2)]

    for lp in params["layers"]:
        epar, npar = lp["edge"], lp["node"]
        W1 = epar["W"][0]
        ew = jnp.stack([W1[2 * H:]] + epar["W"][1:4])
        eb = jnp.stack(epar["b"][0:4] + [epar["g"], epar["be"]])
        t = _pq(nf, jnp.stack([W1[:H], W1[H:2 * H]]))
        gpq0 = _sc_gather(t, idxg[0])
        m0, ef0 = _edge_mlp(gpq0, ef[0], ew, eb)
        gpq1 = _sc_gather(t, idxg[1])
        pa = _sc_scatter(m0, dsti_s[0], zeros)
        m1, ef1 = _edge_mlp(gpq1, ef[1], ew, eb)
        pb = _sc_scatter(m1, dsti_s[1], zeros)
        ef = [ef0, ef1]
        Wn1 = npar["W"][0]
        nf = _node_mlp(
            nf, pa, pb,
            jnp.stack([Wn1[:H], Wn1[H:]] + npar["W"][1:4]),
            jnp.stack(npar["b"][0:4] + [npar["g"], npar["be"]]))

    dp = params["dec"]
    return _dec(nf, jnp.stack(dp["W"][0:3]), jnp.stack(dp["b"][0:3]),
                dp["W"][3], dp["b"][3].reshape(1, 2))


# speedup vs baseline: 4.4838x; 1.0110x over previous
import functools

import jax
import jax.numpy as jnp
from jax import lax
from jax.experimental import pallas as pl
from jax.experimental.pallas import tpu as pltpu
from jax.experimental.pallas import tpu_sc as plsc

NN = 10000
EE = 320000
H = 128
NB = 2000
CS = 128
EPAD = 327680
NE2 = EPAD // 2
EB = 2048
NCORE = 2
NSUB = 16
NW = NCORE * NSUB
NCW = NE2 // NSUB // CS
NCS = NE2 // NW // CS
NPAD = 10240
NPS = NPAD // NSUB
GB = 8
GBG = 16


def _ln(h, g, be):
    mu = jnp.mean(h, axis=-1, keepdims=True)
    d = h - mu
    var = jnp.mean(d * d, axis=-1, keepdims=True)
    return d * lax.rsqrt(var + 1e-5) * g[None, :] + be[None, :]


def _dot(a, b):
    return jnp.dot(a, b, preferred_element_type=jnp.float32)


def _enc_node_body(x_ref, pos_ref, e0_ref, w0b_ref, w_ref, b_ref, o_ref):
    B = b_ref[...]
    oh = (x_ref[...] == lax.broadcasted_iota(jnp.int32, (1, 9), 1)).astype(jnp.float32)
    h = _dot(oh, e0_ref[...]) + _dot(pos_ref[...], w0b_ref[...]) + B[0][None, :]
    h = jnp.maximum(h, 0)
    W = w_ref[...]
    for i in range(3):
        h = _dot(h, W[i]) + B[i + 1][None, :]
        if i < 2:
            h = jnp.maximum(h, 0)
    o_ref[...] = _ln(h, B[4], B[5])


def _enc_edge_body(ea_ref, w0_ref, w_ref, b_ref, o_ref):
    B = b_ref[...]
    h = _dot(ea_ref[...], w0_ref[...]) + B[0][None, :]
    h = jnp.maximum(h, 0)
    W = w_ref[...]
    for i in range(3):
        h = _dot(h, W[i]) + B[i + 1][None, :]
        if i < 2:
            h = jnp.maximum(h, 0)
    o_ref[...] = _ln(h, B[4], B[5])


def _pq_body(nf_ref, w_ref, t_ref):
    t_ref[...] = _dot(nf_ref[...], w_ref[0])


def _edge_mlp_body(gp_ref, gq_ref, ef_ref, w_ref, b_ref, m_ref, efo_ref):
    ef = ef_ref[...]
    W = w_ref[...]
    B = b_ref[...]
    h = gp_ref[0] + gq_ref[0] + _dot(ef, W[0]) + B[0][None, :]
    h = jnp.maximum(h, 0)
    for i in range(1, 4):
        h = _dot(h, W[i]) + B[i][None, :]
        if i < 3:
            h = jnp.maximum(h, 0)
    m = _ln(h, B[4], B[5])
    m_ref[...] = m
    efo_ref[...] = m + ef


def _node_mlp_body(nf_ref, pa_ref, pb_ref, w_ref, b_ref, o_ref):
    nf = nf_ref[...]
    ag = pa_ref[0] + pa_ref[1] + pb_ref[0] + pb_ref[1]
    W = w_ref[...]
    B = b_ref[...]
    h = _dot(nf, W[0]) + _dot(ag, W[1]) + B[0][None, :]
    h = jnp.maximum(h, 0)
    for i in range(1, 4):
        h = _dot(h, W[i + 1]) + B[i][None, :]
        if i < 3:
            h = jnp.maximum(h, 0)
    o_ref[...] = _ln(h, B[4], B[5]) + nf


def _dec_body(nf_ref, w_ref, b_ref, w3_ref, b3_ref, o_ref):
    h = nf_ref[...]
    W = w_ref[...]
    B = b_ref[...]
    for i in range(3):
        h = jnp.maximum(_dot(h, W[i]) + B[i][None, :], 0)
    o_ref[...] = _dot(h, w3_ref[...]) + b3_ref[...]


def _full(shape):
    nd = len(shape)
    return pl.BlockSpec(shape, lambda i, _nd=nd: (0,) * _nd)


def _rows(block, width):
    return pl.BlockSpec((block, width), lambda i: (i, 0))


def _enc_node(x2d, pos, e0, w0b, w, b):
    return pl.pallas_call(
        _enc_node_body,
        grid=(NN // NB,),
        in_specs=[_rows(NB, 1), _rows(NB, 6), _full((9, H)), _full((6, H)),
                  _full((3, H, H)), _full((6, H))],
        out_specs=_rows(NB, H),
        out_shape=jax.ShapeDtypeStruct((NN, H), jnp.float32),
    )(x2d, pos, e0, w0b, w, b)


def _enc_edge(ea, w0, w, b):
    return pl.pallas_call(
        _enc_edge_body,
        grid=(NE2 // EB,),
        in_specs=[_rows(EB, 3), _full((3, H)), _full((3, H, H)), _full((6, H))],
        out_specs=_rows(EB, H),
        out_shape=jax.ShapeDtypeStruct((NE2, H), jnp.float32),
    )(ea, w0, w, b)


def _pq(nf, w):
    return pl.pallas_call(
        _pq_body,
        grid=(2, NN // NB),
        in_specs=[pl.BlockSpec((NB, H), lambda j, i: (i, 0)),
                  pl.BlockSpec((1, H, H), lambda j, i: (j, 0, 0))],
        out_specs=pl.BlockSpec((NB, H), lambda j, i: (j * (NN // NB) + i, 0)),
        out_shape=jax.ShapeDtypeStruct((2 * NN, H), jnp.float32),
    )(nf, w)


def _edge_mlp(gpq, ef, w, b):
    gp_spec = pl.BlockSpec((1, EB, H), lambda i: (0, i, 0))
    gq_spec = pl.BlockSpec((1, EB, H), lambda i: (1, i, 0))
    return pl.pallas_call(
        _edge_mlp_body,
        grid=(NE2 // EB,),
        in_specs=[gp_spec, gq_spec, _rows(EB, H),
                  _full((4, H, H)), _full((6, H))],
        out_specs=[_rows(EB, H), _rows(EB, H)],
        out_shape=[jax.ShapeDtypeStruct((NE2, H), jnp.float32),
                   jax.ShapeDtypeStruct((NE2, H), jnp.float32)],
    )(gpq, gpq, ef, w, b)


def _node_mlp(nf, pa, pb, w, b):
    pspec = pl.BlockSpec((2, NB, H), lambda i: (0, i, 0))
    return pl.pallas_call(
        _node_mlp_body,
        grid=(NN // NB,),
        in_specs=[_rows(NB, H), pspec, pspec, _full((5, H, H)), _full((6, H))],
        out_specs=_rows(NB, H),
        out_shape=jax.ShapeDtypeStruct((NN, H), jnp.float32),
    )(nf, pa, pb, w, b)


def _dec(nf, w, b, w3, b3):
    return pl.pallas_call(
        _dec_body,
        grid=(NN // NB,),
        in_specs=[_rows(NB, H), _full((3, H, H)), _full((3, H)),
                  _full((H, 2)), _full((1, 2))],
        out_specs=_rows(NB, 2),
        out_shape=jax.ShapeDtypeStruct((NN, 2), jnp.float32),
    )(nf, w, b, w3, b3)


def _mesh():
    return plsc.VectorSubcoreMesh(core_axis_name="c", subcore_axis_name="s")


def _sc_gather(t, idxg):
    @functools.partial(
        pl.kernel,
        mesh=_mesh(),
        out_type=jax.ShapeDtypeStruct((2, NE2, H), jnp.float32),
        scratch_types=[pltpu.VMEM((GBG, CS), jnp.int32),
                       pltpu.VMEM((CS, H), jnp.float32),
                       pltpu.VMEM((CS, H), jnp.float32),
                       pltpu.VMEM_SHARED((NN, H), jnp.float32),
                       pltpu.SemaphoreType.DMA,
                       pltpu.SemaphoreType.DMA,
                       pltpu.SemaphoreType.DMA,
                       pltpu.SemaphoreType.DMA],
    )
    def k(t_hbm, ix_hbm, out_hbm, ixv, buf0, buf1, table, g0, g1, w0, w1):
        c = lax.axis_index("c")
        s = lax.axis_index("s")

        @pl.when(s == 0)
        def _():
            pltpu.sync_copy(t_hbm.at[pl.ds(c * NN, NN)], table)

        plsc.subcore_barrier()
        obase = s * NCW * CS
        bufs = (buf0, buf1)
        gsem = (g0, g1)
        wsem = (w0, w1)

        @pl.loop(0, NCW, step=GBG)
        def _(cb):
            pltpu.sync_copy(ix_hbm.at[c].at[s].at[pl.ds(cb, GBG)], ixv)
            hg = [None, None]
            hw = [None, None]
            hg[0] = pltpu.async_copy(table.at[ixv.at[0]], bufs[0], gsem[0])
            for j in range(GBG):
                b = j % 2
                if j + 1 < GBG:
                    if hw[1 - b] is not None:
                        hw[1 - b].wait()
                    hg[1 - b] = pltpu.async_copy(
                        table.at[ixv.at[j + 1]], bufs[1 - b], gsem[1 - b])
                hg[b].wait()
                hw[b] = pltpu.async_copy(
                    bufs[b],
                    out_hbm.at[c].at[pl.ds(obase + (cb + j) * CS, CS)],
                    wsem[b])
            hw[0].wait()
            hw[1].wait()

    return k(t, idxg)


def _sc_scatter(m, dsti, zeros):
    @functools.partial(
        pl.kernel,
        mesh=_mesh(),
        out_type=jax.ShapeDtypeStruct((NCORE, NPAD, H), jnp.float32),
        scratch_types=[pltpu.VMEM((GB, CS), jnp.int32),
                       pltpu.VMEM((CS, H), jnp.float32),
                       pltpu.VMEM((CS, H), jnp.float32),
                       pltpu.VMEM_SHARED((NPAD, H), jnp.float32),
                       pltpu.SemaphoreType.DMA,
                       pltpu.SemaphoreType.DMA,
                       pltpu.SemaphoreType.DMA,
                       pltpu.SemaphoreType.DMA],
    )
    def k(m_hbm, di_hbm, z_hbm, out_hbm, ixv, buf0, buf1, shared, r0, r1,
          a0, a1):
        c = lax.axis_index("c")
        s = lax.axis_index("s")
        wid = s * NCORE + c
        base = wid * NCS * CS

        pltpu.sync_copy(z_hbm.at[pl.ds(s * NPS, NPS)],
                        shared.at[pl.ds(s * NPS, NPS)])
        plsc.subcore_barrier()
        bufs = (buf0, buf1)
        rsem = (r0, r1)
        asem = (a0, a1)

        @pl.loop(0, NCS, step=GB)
        def _(cb):
            pltpu.sync_copy(di_hbm.at[wid].at[pl.ds(cb, GB)], ixv)
            hr = [None, None]
            ha = [None, None]
            hr[0] = pltpu.async_copy(
                m_hbm.at[pl.ds(base + cb * CS, CS)], bufs[0], rsem[0])
            for j in range(GB):
                b = j % 2
                if j + 1 < GB:
                    if ha[1 - b] is not None:
                        ha[1 - b].wait()
                    hr[1 - b] = pltpu.async_copy(
                        m_hbm.at[pl.ds(base + (cb + j + 1) * CS, CS)],
                        bufs[1 - b], rsem[1 - b])
                hr[b].wait()
                ha[b] = pltpu.async_copy(bufs[b], shared.at[ixv.at[j]],
                                         asem[b], add=True)
            ha[0].wait()
            ha[1].wait()

        plsc.subcore_barrier()
        pltpu.sync_copy(shared.at[pl.ds(s * NPS, NPS)],
                        out_hbm.at[c].at[pl.ds(s * NPS, NPS)])

    return k(m, dsti, zeros)


def kernel(x, pos, edge_attr, edge_index, params):
    f32 = jnp.float32
    x2d = x.reshape(NN, 1).astype(jnp.int32)
    pos = pos.astype(f32)
    npadE = EPAD - EE
    dst = edge_index[1].astype(jnp.int32)
    src_ = edge_index[0].astype(jnp.int32)
    dst_p = jnp.pad(dst, (0, npadE))
    src_p = jnp.pad(src_, (0, npadE))
    idxg = [jnp.stack([dst_p[h * NE2:(h + 1) * NE2].reshape(NSUB, NCW, CS),
                       src_p[h * NE2:(h + 1) * NE2].reshape(NSUB, NCW, CS)])
            for h in range(2)]
    padtgt = NN + (jnp.arange(npadE, dtype=jnp.int32) % (NPAD - NN))
    dst_s = jnp.concatenate([dst, padtgt])
    dsti_s = [dst_s[h * NE2:(h + 1) * NE2].reshape(NW, NCS, CS)
              for h in range(2)]
    ea_p = jnp.pad(edge_attr.astype(f32), ((0, npadE), (0, 0)))
    zeros = jnp.zeros((NPAD, H), f32)

    ni = params["node_in"]
    e0 = _dot(params["emb"], ni["W"][0][:16])
    nf = _enc_node(x2d, pos, e0, ni["W"][0][16:22],
                   jnp.stack(ni["W"][1:4]),
                   jnp.stack(ni["b"][0:4] + [ni["g"], ni["be"]]))

    ei = params["edge_in"]
    ew0, eww, ewb = (ei["W"][0], jnp.stack(ei["W"][1:4]),
                     jnp.stack(ei["b"][0:4] + [ei["g"], ei["be"]]))
    ef = [_enc_edge(ea_p[h * NE2:(h + 1) * NE2], ew0, eww, ewb)
          for h in range(2)]

    for lp in params["layers"]:
        epar, npar = lp["edge"], lp["node"]
        W1 = epar["W"][0]
        ew = jnp.stack([W1[2 * H:]] + epar["W"][1:4])
        eb = jnp.stack(epar["b"][0:4] + [epar["g"], epar["be"]])
        t = _pq(nf, jnp.stack([W1[:H], W1[H:2 * H]]))
        gpq0 = _sc_gather(t, idxg[0])
        m0, ef0 = _edge_mlp(gpq0, ef[0], ew, eb)
        gpq1 = _sc_gather(t, idxg[1])
        pa = _sc_scatter(m0, dsti_s[0], zeros)
        m1, ef1 = _edge_mlp(gpq1, ef[1], ew, eb)
        pb = _sc_scatter(m1, dsti_s[1], zeros)
        ef = [ef0, ef1]
        Wn1 = npar["W"][0]
        nf = _node_mlp(
            nf, pa, pb,
            jnp.stack([Wn1[:H], Wn1[H:]] + npar["W"][1:4]),
            jnp.stack(npar["b"][0:4] + [npar["g"], npar["be"]]))

    dp = params["dec"]
    return _dec(nf, jnp.stack(dp["W"][0:3]), jnp.stack(dp["b"][0:3]),
                dp["W"][3], dp["b"][3].reshape(1, 2))
